# baseline (speedup 1.0000x reference)
import functools

import jax
import jax.numpy as jnp
from jax import lax
from jax.experimental import pallas as pl
from jax.experimental.pallas import tpu as pltpu
from jax.experimental.pallas import tpu_sc as plsc

N = 10000
E = 320000
IN_DIM = 128
HID = 64
LAT = 16
NF = N // 2
FD = 2 * HID

NC = 2
NS = 16
NW = NC * NS
EPW = E // NW
WIN = 128
NWIN = EPW // WIN
TAIL = EPW - NWIN * WIN
PIPE = 6
NBLK = NWIN // PIPE
SLICE = N // NS
NPADD = 10240
DSLICE = NPADD // NS

_mesh = plsc.VectorSubcoreMesh(core_axis_name="c", subcore_axis_name="s")
_sc_params = pltpu.CompilerParams(use_tc_tiling_on_sc=False)


@functools.partial(
    pl.kernel,
    out_type=(
        jax.ShapeDtypeStruct((NC, NPADD), jnp.float32),
        jax.ShapeDtypeStruct((NC, NPADD), jnp.float32),
    ),
    mesh=_mesh,
    compiler_params=_sc_params,
    scratch_types=[
        pltpu.VMEM((EPW,), jnp.int32),
        pltpu.VMEM((EPW,), jnp.int32),
        pltpu.VMEM((WIN,), jnp.float32),
        pltpu.VMEM((DSLICE,), jnp.float32),
        pltpu.VMEM_SHARED((NPADD,), jnp.float32),
        pltpu.VMEM_SHARED((NPADD,), jnp.float32),
    ]
    + [pltpu.SemaphoreType.DMA] * (2 * PIPE),
)
def _deg_kernel(src_hbm, dst_hbm, dout_hbm, din_hbm,
                src_v, dst_v, ones_v, zero_v, dout_sp, din_sp, *sems):
    c = lax.axis_index("c")
    s = lax.axis_index("s")
    w = c * NS + s
    for j in range(DSLICE // 16):
        zero_v[pl.ds(16 * j, 16)] = jnp.zeros((16,), jnp.float32)
    for j in range(WIN // 16):
        ones_v[pl.ds(16 * j, 16)] = jnp.ones((16,), jnp.float32)
    pltpu.sync_copy(zero_v, dout_sp.at[pl.ds(s * DSLICE, DSLICE)])
    pltpu.sync_copy(zero_v, din_sp.at[pl.ds(s * DSLICE, DSLICE)])
    pltpu.sync_copy(src_hbm.at[w], src_v)
    pltpu.sync_copy(dst_hbm.at[w], dst_v)
    plsc.subcore_barrier()
    pltpu.sync_copy(ones_v.at[pl.ds(0, TAIL)],
                    dout_sp.at[src_v.at[pl.ds(NWIN * WIN, TAIL)]], add=True)
    pltpu.sync_copy(ones_v.at[pl.ds(0, TAIL)],
                    din_sp.at[dst_v.at[pl.ds(NWIN * WIN, TAIL)]], add=True)

    def body(k, carry):
        ds = []
        for b in range(PIPE):
            j = k * PIPE + b
            ds.append(pltpu.async_copy(
                ones_v, dout_sp.at[src_v.at[pl.ds(j * WIN, WIN)]],
                sems[b], add=True))
            ds.append(pltpu.async_copy(
                ones_v, din_sp.at[dst_v.at[pl.ds(j * WIN, WIN)]],
                sems[PIPE + b], add=True))
        for d in ds:
            d.wait()
        return carry

    lax.fori_loop(0, NBLK, body, 0)
    plsc.subcore_barrier()
    pltpu.sync_copy(dout_sp.at[pl.ds(s * DSLICE, DSLICE)],
                    dout_hbm.at[c, pl.ds(s * DSLICE, DSLICE)])
    pltpu.sync_copy(din_sp.at[pl.ds(s * DSLICE, DSLICE)],
                    din_hbm.at[c, pl.ds(s * DSLICE, DSLICE)])


@functools.partial(
    pl.kernel,
    out_type=jax.ShapeDtypeStruct((NC, N, HID), jnp.float32),
    mesh=_mesh,
    compiler_params=_sc_params,
    scratch_types=[
        pltpu.VMEM((EPW,), jnp.int32),
        pltpu.VMEM((EPW,), jnp.int32),
        pltpu.VMEM((TAIL, HID), jnp.float32),
        pltpu.VMEM_SHARED((N, HID), jnp.float32),
    ]
    + [pltpu.VMEM((WIN, HID), jnp.float32)] * PIPE
    + [pltpu.SemaphoreType.DMA] * PIPE,
)
def _scatter_kernel(t_hbm, src_hbm, dst_hbm, zeros_hbm,
                    out_hbm, src_v, dst_v, tail_buf, acc_sp, *bufs_and_sems):
    bufs = bufs_and_sems[:PIPE]
    gsem = bufs_and_sems[PIPE:]
    c = lax.axis_index("c")
    s = lax.axis_index("s")
    w = c * NS + s
    pltpu.sync_copy(zeros_hbm.at[pl.ds(s * SLICE, SLICE)],
                    acc_sp.at[pl.ds(s * SLICE, SLICE)])
    pltpu.sync_copy(src_hbm.at[w], src_v)
    pltpu.sync_copy(dst_hbm.at[w], dst_v)
    plsc.subcore_barrier()
    pltpu.sync_copy(t_hbm.at[src_v.at[pl.ds(NWIN * WIN, TAIL)]], tail_buf)
    pltpu.sync_copy(tail_buf, acc_sp.at[dst_v.at[pl.ds(NWIN * WIN, TAIL)]],
                    add=True)
    for b in range(PIPE):
        pltpu.async_copy(t_hbm.at[src_v.at[pl.ds(b * WIN, WIN)]],
                         bufs[b], gsem[b])

    def body(k, carry):
        for b in range(PIPE):
            j = k * PIPE + b
            pltpu.make_async_copy(
                t_hbm.at[src_v.at[pl.ds(j * WIN, WIN)]], bufs[b],
                gsem[b]).wait()
            pltpu.sync_copy(bufs[b],
                            acc_sp.at[dst_v.at[pl.ds(j * WIN, WIN)]],
                            add=True)

            @pl.when(j + PIPE < NWIN)
            def _():
                pltpu.async_copy(
                    t_hbm.at[src_v.at[pl.ds((j + PIPE) * WIN, WIN)]],
                    bufs[b], gsem[b])
        return carry

    lax.fori_loop(0, NBLK, body, 0)
    plsc.subcore_barrier()
    pltpu.sync_copy(acc_sp.at[pl.ds(s * SLICE, SLICE)],
                    out_hbm.at[c, pl.ds(s * SLICE, SLICE)])


def _tc_mm1(xf, W1d, doutf):
    def body(x_ref, w_ref, dg_ref, o_ref):
        o_ref[...] = jnp.dot(x_ref[...], w_ref[...],
                             preferred_element_type=jnp.float32) * dg_ref[...]

    return pl.pallas_call(
        body,
        out_shape=jax.ShapeDtypeStruct((NF, FD), jnp.float32),
    )(xf, W1d, doutf)


def _tc_mm2(aggf, dinf, b1d, W2d, doutf):
    def body(a_ref, din_ref, b_ref, w_ref, dout_ref, o_ref):
        h = jnp.maximum(a_ref[...] * din_ref[...] + b_ref[...], 0.0)
        o_ref[...] = jnp.dot(h, w_ref[...],
                             preferred_element_type=jnp.float32) * dout_ref[...]

    return pl.pallas_call(
        body,
        out_shape=jax.ShapeDtypeStruct((NF, FD), jnp.float32),
    )(aggf, dinf, b1d, W2d, doutf)


def _tc_head(aggf, dinf, b2d, Wmu, bmu_row, Wlv, blv_row):
    def body(a_ref, din_ref, b_ref, wmu_ref, bmu_ref, wlv_ref, blv_ref, o_ref):
        h = jnp.maximum(a_ref[...] * din_ref[...] + b_ref[...], 0.0)
        hs = jnp.sum(h, axis=0, keepdims=True)
        hg = (hs[:, :HID] + hs[:, HID:]) * (1.0 / N)
        mu = jnp.dot(hg, wmu_ref[...], preferred_element_type=jnp.float32)
        lv = jnp.dot(hg, wlv_ref[...], preferred_element_type=jnp.float32)
        o_ref[...] = jnp.concatenate([mu + bmu_ref[...], lv + blv_ref[...]],
                                     axis=0)

    return pl.pallas_call(
        body,
        out_shape=jax.ShapeDtypeStruct((2, LAT), jnp.float32),
    )(aggf, dinf, b2d, Wmu, bmu_row, Wlv, blv_row)


def kernel(x, edge_index, edge_feat, W1, b1, W2, b2, We, be, Wmu, bmu, Wlv, blv):
    src2 = edge_index[0].reshape(NW, EPW)
    dst2 = edge_index[1].reshape(NW, EPW)
    zeros2d = jnp.zeros((N, HID), jnp.float32)

    W1d = jnp.zeros((2 * IN_DIM, FD), jnp.float32)
    W1d = W1d.at[:IN_DIM, :HID].set(W1).at[IN_DIM:, HID:].set(W1)
    W2d = jnp.zeros((FD, FD), jnp.float32)
    W2d = W2d.at[:HID, :HID].set(W2).at[HID:, HID:].set(W2)
    b1d = jnp.concatenate([b1, b1]).reshape(1, FD)
    b2d = jnp.concatenate([b2, b2]).reshape(1, FD)
    xf = x.reshape(NF, 2 * IN_DIM)

    deg_out2, deg_in2 = _deg_kernel(src2, dst2)
    doutf = lax.rsqrt(jnp.maximum(
        deg_out2[0, :N] + deg_out2[1, :N], 1.0))[:, None]
    doutf = jnp.broadcast_to(doutf, (N, HID)).reshape(NF, FD)
    dinf = lax.rsqrt(jnp.maximum(
        deg_in2[0, :N] + deg_in2[1, :N], 1.0))[:, None]
    dinf = jnp.broadcast_to(dinf, (N, HID)).reshape(NF, FD)

    t1 = _tc_mm1(xf, W1d, doutf).reshape(N, HID)
    acc1 = _scatter_kernel(t1, src2, dst2, zeros2d)
    af1 = acc1.reshape(NC, NF, FD)
    agg1f = af1[0] + af1[1]
    t2 = _tc_mm2(agg1f, dinf, b1d, W2d, doutf).reshape(N, HID)
    acc2 = _scatter_kernel(t2, src2, dst2, zeros2d)
    af2 = acc2.reshape(NC, NF, FD)
    agg2f = af2[0] + af2[1]
    out = _tc_head(agg2f, dinf, b2d,
                   Wmu, bmu.reshape(1, LAT), Wlv, blv.reshape(1, LAT))
    return out[0:1], out[1:2]


# speedup vs baseline: 1.0633x; 1.0633x over previous
import functools

import jax
import jax.numpy as jnp
from jax import lax
from jax.experimental import pallas as pl
from jax.experimental.pallas import tpu as pltpu
from jax.experimental.pallas import tpu_sc as plsc

N = 10000
E = 320000
IN_DIM = 128
HID = 64
LAT = 16
NF = N // 2
FD = 2 * HID

NC = 2
NS = 16
NW = NC * NS
EPW = E // NW
WIN = 128
NWIN = EPW // WIN
TAIL = EPW - NWIN * WIN
PIPE = 6
NBLK = NWIN // PIPE
SLICE = N // NS
NPADD = 10240
DSLICE = NPADD // NS

_mesh = plsc.VectorSubcoreMesh(core_axis_name="c", subcore_axis_name="s")
_sc_params = pltpu.CompilerParams(use_tc_tiling_on_sc=False)


@functools.partial(
    pl.kernel,
    out_type=(
        jax.ShapeDtypeStruct((NC, NPADD), jnp.float32),
        jax.ShapeDtypeStruct((NC, NPADD), jnp.float32),
    ),
    mesh=_mesh,
    compiler_params=_sc_params,
    scratch_types=[
        pltpu.VMEM((EPW,), jnp.int32),
        pltpu.VMEM((EPW,), jnp.int32),
        pltpu.VMEM((WIN,), jnp.float32),
        pltpu.VMEM((DSLICE,), jnp.float32),
        pltpu.VMEM_SHARED((NPADD,), jnp.float32),
        pltpu.VMEM_SHARED((NPADD,), jnp.float32),
    ]
    + [pltpu.SemaphoreType.DMA] * (2 * PIPE),
)
def _deg_kernel(src_hbm, dst_hbm, dout_hbm, din_hbm,
                src_v, dst_v, ones_v, zero_v, dout_sp, din_sp, *sems):
    c = lax.axis_index("c")
    s = lax.axis_index("s")
    w = c * NS + s
    for j in range(DSLICE // 16):
        zero_v[pl.ds(16 * j, 16)] = jnp.zeros((16,), jnp.float32)
    for j in range(WIN // 16):
        ones_v[pl.ds(16 * j, 16)] = jnp.ones((16,), jnp.float32)
    pltpu.sync_copy(zero_v, dout_sp.at[pl.ds(s * DSLICE, DSLICE)])
    pltpu.sync_copy(zero_v, din_sp.at[pl.ds(s * DSLICE, DSLICE)])
    pltpu.sync_copy(src_hbm.at[w], src_v)
    pltpu.sync_copy(dst_hbm.at[w], dst_v)
    plsc.subcore_barrier()
    pltpu.sync_copy(ones_v.at[pl.ds(0, TAIL)],
                    dout_sp.at[src_v.at[pl.ds(NWIN * WIN, TAIL)]], add=True)
    pltpu.sync_copy(ones_v.at[pl.ds(0, TAIL)],
                    din_sp.at[dst_v.at[pl.ds(NWIN * WIN, TAIL)]], add=True)

    def body(k, carry):
        ds = []
        for b in range(PIPE):
            j = k * PIPE + b
            ds.append(pltpu.async_copy(
                ones_v, dout_sp.at[src_v.at[pl.ds(j * WIN, WIN)]],
                sems[b], add=True))
            ds.append(pltpu.async_copy(
                ones_v, din_sp.at[dst_v.at[pl.ds(j * WIN, WIN)]],
                sems[PIPE + b], add=True))
        for d in ds:
            d.wait()
        return carry

    lax.fori_loop(0, NBLK, body, 0)
    plsc.subcore_barrier()
    pltpu.sync_copy(dout_sp.at[pl.ds(s * DSLICE, DSLICE)],
                    dout_hbm.at[c, pl.ds(s * DSLICE, DSLICE)])
    pltpu.sync_copy(din_sp.at[pl.ds(s * DSLICE, DSLICE)],
                    din_hbm.at[c, pl.ds(s * DSLICE, DSLICE)])


@functools.partial(
    pl.kernel,
    out_type=jax.ShapeDtypeStruct((NC, N, HID), jnp.float32),
    mesh=_mesh,
    compiler_params=_sc_params,
    scratch_types=[
        pltpu.VMEM((EPW,), jnp.int32),
        pltpu.VMEM((EPW,), jnp.int32),
        pltpu.VMEM((TAIL, HID), jnp.float32),
        pltpu.VMEM_SHARED((N, HID), jnp.float32),
    ]
    + [pltpu.VMEM((WIN, HID), jnp.float32)] * PIPE
    + [pltpu.SemaphoreType.DMA] * PIPE,
)
def _scatter_kernel(t_hbm, src_hbm, dst_hbm, zeros_hbm,
                    out_hbm, src_v, dst_v, tail_buf, acc_sp, *bufs_and_sems):
    bufs = bufs_and_sems[:PIPE]
    gsem = bufs_and_sems[PIPE:]
    c = lax.axis_index("c")
    s = lax.axis_index("s")
    w = c * NS + s
    pltpu.sync_copy(zeros_hbm.at[pl.ds(s * SLICE, SLICE)],
                    acc_sp.at[pl.ds(s * SLICE, SLICE)])
    pltpu.sync_copy(src_hbm.at[w], src_v)
    pltpu.sync_copy(dst_hbm.at[w], dst_v)
    plsc.subcore_barrier()
    pltpu.sync_copy(t_hbm.at[src_v.at[pl.ds(NWIN * WIN, TAIL)]], tail_buf)
    pltpu.sync_copy(tail_buf, acc_sp.at[dst_v.at[pl.ds(NWIN * WIN, TAIL)]],
                    add=True)
    for b in range(PIPE):
        pltpu.async_copy(t_hbm.at[src_v.at[pl.ds(b * WIN, WIN)]],
                         bufs[b], gsem[b])

    def body(k, carry):
        for b in range(PIPE):
            j = k * PIPE + b
            pltpu.make_async_copy(
                t_hbm.at[src_v.at[pl.ds(j * WIN, WIN)]], bufs[b],
                gsem[b]).wait()
            pltpu.sync_copy(bufs[b],
                            acc_sp.at[dst_v.at[pl.ds(j * WIN, WIN)]],
                            add=True)

            @pl.when(j + PIPE < NWIN)
            def _():
                pltpu.async_copy(
                    t_hbm.at[src_v.at[pl.ds((j + PIPE) * WIN, WIN)]],
                    bufs[b], gsem[b])
        return carry

    lax.fori_loop(0, NBLK, body, 0)
    plsc.subcore_barrier()
    pltpu.sync_copy(acc_sp.at[pl.ds(s * SLICE, SLICE)],
                    out_hbm.at[c, pl.ds(s * SLICE, SLICE)])


def _tc_mm1(xf, W1d, doutf):
    def body(x_ref, w_ref, dg_ref, o_ref):
        o_ref[...] = jnp.dot(x_ref[...], w_ref[...],
                             preferred_element_type=jnp.float32) * dg_ref[...]

    return pl.pallas_call(
        body,
        out_shape=jax.ShapeDtypeStruct((NF, FD), jnp.float32),
    )(xf, W1d, doutf)


def _tc_mm2(aggf, dinf, b1d, W2d, doutf):
    def body(a_ref, din_ref, b_ref, w_ref, dout_ref, o_ref):
        h = jnp.maximum(a_ref[...] * din_ref[...] + b_ref[...], 0.0)
        o_ref[...] = jnp.dot(h, w_ref[...],
                             preferred_element_type=jnp.float32) * dout_ref[...]

    return pl.pallas_call(
        body,
        out_shape=jax.ShapeDtypeStruct((NF, FD), jnp.float32),
    )(aggf, dinf, b1d, W2d, doutf)


def _tc_head(aggf, dinf, b2d, Wmu, bmu_row, Wlv, blv_row):
    def body(a_ref, din_ref, b_ref, wmu_ref, bmu_ref, wlv_ref, blv_ref, o_ref):
        h = jnp.maximum(a_ref[...] * din_ref[...] + b_ref[...], 0.0)
        hs = jnp.sum(h, axis=0, keepdims=True)
        hg = (hs[:, :HID] + hs[:, HID:]) * (1.0 / N)
        mu = jnp.dot(hg, wmu_ref[...], preferred_element_type=jnp.float32)
        lv = jnp.dot(hg, wlv_ref[...], preferred_element_type=jnp.float32)
        o_ref[...] = jnp.concatenate([mu + bmu_ref[...], lv + blv_ref[...]],
                                     axis=0)

    return pl.pallas_call(
        body,
        out_shape=jax.ShapeDtypeStruct((2, LAT), jnp.float32),
    )(aggf, dinf, b2d, Wmu, bmu_row, Wlv, blv_row)


def kernel(x, edge_index, edge_feat, W1, b1, W2, b2, We, be, Wmu, bmu, Wlv, blv):
    src2 = edge_index[0].reshape(NW, EPW)
    dst2 = edge_index[1].reshape(NW, EPW)
    zeros2d = jnp.zeros((N, HID), jnp.float32)

    W1d = jnp.zeros((2 * IN_DIM, FD), jnp.float32)
    W1d = W1d.at[:IN_DIM, :HID].set(W1).at[IN_DIM:, HID:].set(W1)
    W2d = jnp.zeros((FD, FD), jnp.float32)
    W2d = W2d.at[:HID, :HID].set(W2).at[HID:, HID:].set(W2)
    b1d = jnp.concatenate([b1, b1]).reshape(1, FD)
    b2d = jnp.concatenate([b2, b2]).reshape(1, FD)
    xf = x.reshape(NF, 2 * IN_DIM)

    deg_out2, deg_in2 = _deg_kernel(src2, dst2)
    doutf = lax.rsqrt(jnp.maximum(
        deg_out2[0, :N] + deg_out2[1, :N], 1.0))[:, None]
    doutf = jnp.broadcast_to(doutf, (N, HID)).reshape(NF, FD)
    dinf = lax.rsqrt(jnp.maximum(
        deg_in2[0, :N] + deg_in2[1, :N], 1.0))[:, None]
    dinf = jnp.broadcast_to(dinf, (N, HID)).reshape(NF, FD)

    t1 = _tc_mm1(xf, W1d, doutf).reshape(N, HID)
    acc1 = _scatter_kernel(t1, src2, dst2, zeros2d)
    agg1f = (acc1[0] + acc1[1]).reshape(NF, FD)
    t2 = _tc_mm2(agg1f, dinf, b1d, W2d, doutf).reshape(N, HID)
    acc2 = _scatter_kernel(t2, src2, dst2, zeros2d)
    agg2f = (acc2[0] + acc2[1]).reshape(NF, FD)
    out = _tc_head(agg2f, dinf, b2d,
                   Wmu, bmu.reshape(1, LAT), Wlv, blv.reshape(1, LAT))
    return out[0:1], out[1:2]
